# baseline (speedup 1.0000x reference)
import jax
import jax.numpy as jnp
from jax import lax
from jax.experimental import pallas as pl
from jax.experimental.pallas import tpu as pltpu
from jax.experimental.pallas import tpu_sc as plsc

_TOP_K = 256
_ALPHA = 0.05
_PEN = 0.15
_N = 32768
_NS = 16
_CH = _N // _NS
_NV = _CH // 16


def _lane():
    return lax.iota(jnp.int32, 16)


def _key_of(x):
    i = lax.bitcast_convert_type(x, jnp.int32)
    return jnp.where(i >= 0, i, i ^ jnp.int32(0x7FFFFFFF))


def _allsum(v):
    ln = _lane()
    for k in (8, 4, 2, 1):
        v = v + v[ln ^ k]
    return v


def _allmax(v):
    ln = _lane()
    for k in (8, 4, 2, 1):
        v = jnp.maximum(v, v[ln ^ k])
    return v


def _cumsum(v):
    ln = _lane()
    zero = jnp.zeros((16,), v.dtype)
    for k in (1, 2, 4, 8):
        sh = v[jnp.maximum(ln - k, 0)]
        v = v + jnp.where(ln >= k, sh, zero)
    return v


def _mask_lane(splat, b):
    return jnp.where(_lane() == b, splat, jnp.zeros((16,), splat.dtype))


def _lane0(splat):
    return jnp.where(_lane() == 0, splat, jnp.zeros((16,), splat.dtype))


def _board_total(board_ref):
    acc = jnp.zeros((16,), board_ref.dtype)
    for j in range(_NS):
        acc = acc + board_ref[pl.ds(j * 16, 16)]
    return _allsum(acc)


def _sc_body(p_hbm, q_hbm, adj_hbm, pen_hbm,
             pv, qv, kpv, kqv, adjv,
             rowf, rowi, boardf, boardi,
             bh0, bh1, bh2, bh3, bh4, bh5, bh6, bh7,
             bh8, bh9, bh10, bh11, bh12, bh13, bh14, bh15,
             bd_sum_p, bd_sum_q, bd_up, bd_lo):
    t = lax.axis_index("s")
    zeros_i = jnp.zeros((16,), jnp.int32)
    k = jnp.full((16,), _TOP_K, jnp.int32)
    sign = jnp.int32(-2147483648)
    bdh_p = (bh0, bh1, bh2, bh3, bh4, bh5, bh6, bh7)
    bdh_q = (bh8, bh9, bh10, bh11, bh12, bh13, bh14, bh15)

    pltpu.sync_copy(p_hbm.at[pl.ds(t * _CH, _CH)], pv)
    pltpu.sync_copy(q_hbm.at[pl.ds(t * _CH, _CH)], qv)

    plsc.subcore_barrier()

    def pass1(i, carry):
        sp, sq = carry
        x = pv[pl.ds(i * 16, 16)]
        y = qv[pl.ds(i * 16, 16)]
        kpv[pl.ds(i * 16, 16)] = _key_of(x)
        kqv[pl.ds(i * 16, 16)] = _key_of(y)
        return sp + x, sq + y

    sp, sq = lax.fori_loop(0, _NV, pass1,
                           (jnp.zeros((16,), jnp.float32),
                            jnp.zeros((16,), jnp.float32)))

    rowf[...] = _lane0(_allsum(sp))
    pltpu.sync_copy(rowf, bd_sum_p.at[pl.ds(t * 16, 16)])
    rowf[...] = _lane0(_allsum(sq))
    pltpu.sync_copy(rowf, bd_sum_q.at[pl.ds(t * 16, 16)])

    ln = _lane()
    one_i = jnp.ones((16,), jnp.int32)
    prefix_p = jnp.zeros((16,), jnp.int32)
    prefix_q = jnp.zeros((16,), jnp.int32)
    cntgt_p = jnp.zeros((16,), jnp.int32)
    cntgt_q = jnp.zeros((16,), jnp.int32)

    def hist_vec(accs):
        h = jnp.zeros((16,), jnp.int32)
        for a in range(4):
            for f in range(4):
                cnt = (accs[a] >> (8 * f)) & 255
                tot = _allsum(cnt)
                h = h | _mask_lane(tot, a * 4 + f)
        return h

    for r in range(8):
        shift = 28 - 4 * r
        pm = (1 << (4 * r)) - 1

        def count_pass(i, carry):
            ap0, ap1, ap2, ap3, aq0, aq1, aq2, aq3 = carry
            bx = kpv[pl.ds(i * 16, 16)] ^ sign
            by = kqv[pl.ds(i * 16, 16)] ^ sign
            dx = (bx >> shift) & 15
            dy = (by >> shift) & 15
            ohx = jnp.left_shift(one_i, (dx & 3) * 8)
            ohy = jnp.left_shift(one_i, (dy & 3) * 8)
            if r > 0:
                inx = ((bx >> (shift + 4)) & pm) == prefix_p
                iny = ((by >> (shift + 4)) & pm) == prefix_q
                ohx = jnp.where(inx, ohx, zeros_i)
                ohy = jnp.where(iny, ohy, zeros_i)
            gx = dx >> 2
            gy = dy >> 2
            ap0 = ap0 + jnp.where(gx == 0, ohx, zeros_i)
            ap1 = ap1 + jnp.where(gx == 1, ohx, zeros_i)
            ap2 = ap2 + jnp.where(gx == 2, ohx, zeros_i)
            ap3 = ap3 + jnp.where(gx == 3, ohx, zeros_i)
            aq0 = aq0 + jnp.where(gy == 0, ohy, zeros_i)
            aq1 = aq1 + jnp.where(gy == 1, ohy, zeros_i)
            aq2 = aq2 + jnp.where(gy == 2, ohy, zeros_i)
            aq3 = aq3 + jnp.where(gy == 3, ohy, zeros_i)
            return ap0, ap1, ap2, ap3, aq0, aq1, aq2, aq3

        accs = lax.fori_loop(0, _NV, count_pass,
                             tuple(jnp.zeros((16,), jnp.int32)
                                   for _ in range(8)))

        rowi[...] = hist_vec(accs[:4])
        pltpu.sync_copy(rowi, bdh_p[r].at[pl.ds(t * 16, 16)])
        rowi[...] = hist_vec(accs[4:])
        pltpu.sync_copy(rowi, bdh_q[r].at[pl.ds(t * 16, 16)])
        plsc.subcore_barrier()

        def scan_board(board, rem):
            pltpu.sync_copy(board, boardi)
            comb = jnp.zeros((16,), jnp.int32)
            for j in range(_NS):
                comb = comb + boardi[pl.ds(j * 16, 16)]
            sfx = lax.rev(_cumsum(lax.rev(comb, (0,))), (0,))
            ge = sfx >= rem
            b = _allmax(jnp.where(ge, ln, jnp.full((16,), -1, jnp.int32)))
            gt = _allmax(jnp.where(ge, jnp.zeros((16,), jnp.int32), sfx))
            return b, gt

        b_p, gt_p = scan_board(bdh_p[r], k - cntgt_p)
        b_q, gt_q = scan_board(bdh_q[r], k - cntgt_q)
        cntgt_p = cntgt_p + gt_p
        cntgt_q = cntgt_q + gt_q
        prefix_p = jnp.left_shift(prefix_p, 4) | b_p
        prefix_q = jnp.left_shift(prefix_q, 4) | b_q

    t_p = prefix_p ^ sign
    t_q = prefix_q ^ sign
    r_need_p = k - cntgt_p
    r_need_q = k - cntgt_q

    ones_i = jnp.ones((16,), jnp.int32)
    ln16 = _lane()

    def eq_before(board, final_digit):
        pltpu.sync_copy(board, boardi)
        acc = zeros_i
        for j in range(_NS):
            row = boardi[pl.ds(j * 16, 16)]
            row = jnp.where(ln16 == final_digit, row, zeros_i)
            acc = acc + jnp.where(j < t, row, zeros_i)
        return _allsum(acc)

    eq_before_p = eq_before(bdh_p[7], prefix_p & 15)
    eq_before_q = eq_before(bdh_q[7], prefix_q & 15)
    r_local_p = r_need_p - eq_before_p
    r_local_q = r_need_q - eq_before_q

    pltpu.sync_copy(bd_sum_p, boardf)
    total_p = _board_total(boardf)
    pltpu.sync_copy(bd_sum_q, boardf)
    total_q = _board_total(boardf)
    blend_total = (total_p * jnp.float32(1.0 - _ALPHA)
                   + total_q * jnp.float32(_ALPHA))
    inv_s = jnp.float32(1.0) / (blend_total + jnp.float32(1e-12))

    def final_pass(i, carry):
        up, lo, cp, cq = carry
        x = pv[pl.ds(i * 16, 16)]
        y = qv[pl.ds(i * 16, 16)]
        kx = kpv[pl.ds(i * 16, 16)]
        ky = kqv[pl.ds(i * 16, 16)]
        eqx = jnp.where(kx == t_p, ones_i, zeros_i)
        eqy = jnp.where(ky == t_q, ones_i, zeros_i)
        rankx = cp + _cumsum(eqx) - eqx
        ranky = cq + _cumsum(eqy) - eqy
        curr = (jnp.where(kx > t_p, ones_i, zeros_i)
                | (eqx & jnp.where(rankx < r_local_p, ones_i, zeros_i)))
        prev = (jnp.where(ky > t_q, ones_i, zeros_i)
                | (eqy & jnp.where(ranky < r_local_q, ones_i, zeros_i)))
        zf = jnp.zeros((16,), jnp.float32)
        up = up + jnp.where(curr > prev, x, zf)
        lo = lo + jnp.where(prev > curr, y, zf)
        blend = x * jnp.float32(1.0 - _ALPHA) + y * jnp.float32(_ALPHA)
        adjv[pl.ds(i * 16, 16)] = blend * inv_s
        return up, lo, cp + _allsum(eqx), cq + _allsum(eqy)

    up, lo, _, _ = lax.fori_loop(
        0, _NV, final_pass,
        (jnp.zeros((16,), jnp.float32), jnp.zeros((16,), jnp.float32),
         jnp.zeros((16,), jnp.int32), jnp.zeros((16,), jnp.int32)))

    pltpu.sync_copy(adjv, adj_hbm.at[pl.ds(t * _CH, _CH)])

    rowf[...] = _lane0(_allsum(up))
    pltpu.sync_copy(rowf, bd_up.at[pl.ds(t * 16, 16)])
    rowf[...] = _lane0(_allsum(lo))
    pltpu.sync_copy(rowf, bd_lo.at[pl.ds(t * 16, 16)])
    plsc.subcore_barrier()

    @pl.when(t == 0)
    def _():
        pltpu.sync_copy(bd_up, boardf)
        upper = _board_total(boardf)
        pltpu.sync_copy(bd_lo, boardf)
        lower = _board_total(boardf)
        rowf[...] = jnp.float32(_PEN) * (upper + lower)
        pltpu.sync_copy(rowf, pen_hbm)


_SC_OUT = (
    jax.ShapeDtypeStruct((_N,), jnp.float32),
    jax.ShapeDtypeStruct((16,), jnp.float32),
)

_SC_SCRATCH = [
        pltpu.VMEM((_CH,), jnp.float32),
        pltpu.VMEM((_CH,), jnp.float32),
        pltpu.VMEM((_CH,), jnp.int32),
        pltpu.VMEM((_CH,), jnp.int32),
        pltpu.VMEM((_CH,), jnp.float32),
        pltpu.VMEM((16,), jnp.float32),
        pltpu.VMEM((16,), jnp.int32),
        pltpu.VMEM((_NS * 16,), jnp.float32),
        pltpu.VMEM((_NS * 16,), jnp.int32),
        pltpu.VMEM_SHARED((_NS * 16,), jnp.int32),
        pltpu.VMEM_SHARED((_NS * 16,), jnp.int32),
        pltpu.VMEM_SHARED((_NS * 16,), jnp.int32),
        pltpu.VMEM_SHARED((_NS * 16,), jnp.int32),
        pltpu.VMEM_SHARED((_NS * 16,), jnp.int32),
        pltpu.VMEM_SHARED((_NS * 16,), jnp.int32),
        pltpu.VMEM_SHARED((_NS * 16,), jnp.int32),
        pltpu.VMEM_SHARED((_NS * 16,), jnp.int32),
        pltpu.VMEM_SHARED((_NS * 16,), jnp.int32),
        pltpu.VMEM_SHARED((_NS * 16,), jnp.int32),
        pltpu.VMEM_SHARED((_NS * 16,), jnp.int32),
        pltpu.VMEM_SHARED((_NS * 16,), jnp.int32),
        pltpu.VMEM_SHARED((_NS * 16,), jnp.int32),
        pltpu.VMEM_SHARED((_NS * 16,), jnp.int32),
        pltpu.VMEM_SHARED((_NS * 16,), jnp.int32),
        pltpu.VMEM_SHARED((_NS * 16,), jnp.int32),
        pltpu.VMEM_SHARED((_NS * 16,), jnp.float32),
        pltpu.VMEM_SHARED((_NS * 16,), jnp.float32),
        pltpu.VMEM_SHARED((_NS * 16,), jnp.float32),
        pltpu.VMEM_SHARED((_NS * 16,), jnp.float32),
]

_sc_call = pl.kernel(
    _sc_body,
    out_type=_SC_OUT,
    mesh=plsc.VectorSubcoreMesh(
        core_axis_name="c", subcore_axis_name="s", num_cores=1, num_subcores=16),
    scratch_types=_SC_SCRATCH,
)


@jax.jit
def kernel(probs, prev_probs):
    adj, pen = _sc_call(probs, prev_probs)
    return adj, jax.lax.stop_gradient(pen[0])
